# Initial kernel scaffold; baseline (speedup 1.0000x reference)
import jax
import jax.numpy as jnp
from jax.experimental import pallas as pl


def kernel(x, adj, W_enc, b_enc, W_fc, b_fc):
    raise NotImplementedError("write your pallas kernel here")


# speedup vs baseline: 1.6591x; 1.6591x over previous
import functools
import math

import jax
import jax.numpy as jnp
from jax.experimental import pallas as pl
from jax.experimental.pallas import tpu as pltpu

LAMDA = 0.5
TAU = 0.5
P = 0.5
_INV_TAU = 1.0 / TAU
_INV_KEEP = 1.0 / (1.0 - P)


def _enc_body(x_ref, w_ref, b_ref, out_ref):
    out_ref[...] = jax.lax.dot_general(
        x_ref[...], w_ref[...], (((1,), (1,)), ((), ())),
        preferred_element_type=jnp.float32) + b_ref[...]


def _prop_body(adj_ref, hfull_ref, hrow_ref, out_ref, *, beta):
    hi = jax.lax.dot_general(
        adj_ref[...], hfull_ref[...], (((1,), (0,)), ((), ())),
        preferred_element_type=jnp.float32)
    out_ref[...] = beta * hrow_ref[...] + (1.0 - beta) * hi


def _layer2_body(adj_ref, c0full_ref, c0row_ref, u1_ref, u2_ref, wfc_ref,
                 bfc_ref, logp_ref, z1n_ref, z2n_ref, *, beta):
    hi = jax.lax.dot_general(
        adj_ref[...], c0full_ref[...], (((1,), (0,)), ((), ())),
        preferred_element_type=jnp.float32)
    c0row = c0row_ref[...]
    cur1 = beta * c0row + (1.0 - beta) * hi

    z1 = jnp.where(u1_ref[...] >= P, cur1 * _INV_KEEP, 0.0)
    z2 = jnp.where(u2_ref[...] >= P, c0row * _INV_KEEP, 0.0)
    n1 = jnp.sqrt(jnp.sum(z1 * z1, axis=1, keepdims=True))
    n2 = jnp.sqrt(jnp.sum(z2 * z2, axis=1, keepdims=True))
    z1n_ref[...] = z1 / jnp.maximum(n1, 1e-12)
    z2n_ref[...] = z2 / jnp.maximum(n2, 1e-12)

    fc = jax.lax.dot_general(
        cur1, wfc_ref[...], (((1,), (1,)), ((), ())),
        preferred_element_type=jnp.float32) + bfc_ref[...]
    m = jnp.max(fc, axis=1, keepdims=True)
    sh = fc - m
    logp_ref[...] = sh - jnp.log(jnp.sum(jnp.exp(sh), axis=1, keepdims=True))


def _loss_body(z1row_ref, z2row_ref, z1full_ref, z2full_ref, out_ref):
    i = pl.program_id(0)
    a = z1row_ref[...]
    s1 = jax.lax.dot_general(
        a, z1full_ref[...], (((1,), (1,)), ((), ())),
        preferred_element_type=jnp.float32)
    s2 = jax.lax.dot_general(
        a, z2full_ref[...], (((1,), (1,)), ((), ())),
        preferred_element_type=jnp.float32)
    rs = (jnp.sum(jnp.exp(s1 * _INV_TAU), axis=1, keepdims=True)
          + jnp.sum(jnp.exp(s2 * _INV_TAU), axis=1, keepdims=True))
    d_refl = jnp.exp(jnp.sum(a * a, axis=1, keepdims=True) * _INV_TAU)
    dotb = jnp.sum(a * z2row_ref[...], axis=1, keepdims=True)
    d_btw = jnp.exp(dotb * _INV_TAU)
    ct = jnp.log(rs - d_refl - d_btw) - dotb * _INV_TAU
    bs = jnp.sum(ct)

    @pl.when(i == 0)
    def _():
        out_ref[0, 0] = 0.0

    out_ref[0, 0] += bs


def kernel(x, adj, W_enc, b_enc, W_fc, b_fc):
    x2 = jnp.squeeze(x, 0)
    n, nfeat = x2.shape
    hid = W_enc.shape[0]
    ncls = W_fc.shape[0]

    bm = 200 if n % 200 == 0 else n
    grid = (n // bm,)
    beta1 = math.log(LAMDA / 1.0 + 1.0)
    beta2 = math.log(LAMDA / 2.0 + 1.0)

    b_enc2 = b_enc.reshape(1, hid)
    b_fc2 = b_fc.reshape(1, ncls)

    dkey = jax.random.key(1)
    k1, k2 = jax.random.split(jax.random.fold_in(dkey, 1))
    u1 = jax.random.uniform(k1, (n, hid), dtype=jnp.float32)
    u2 = jax.random.uniform(k2, (n, hid), dtype=jnp.float32)

    row_blk = lambda w: pl.BlockSpec((bm, w), lambda i: (i, 0))
    full_blk = lambda a: pl.BlockSpec(a.shape, lambda i: (0, 0))

    bm_e = 2000 if n % 2000 == 0 else bm
    h0 = pl.pallas_call(
        _enc_body,
        grid=(n // bm_e,),
        in_specs=[pl.BlockSpec((bm_e, nfeat), lambda i: (i, 0)),
                  full_blk(W_enc), full_blk(b_enc2)],
        out_specs=pl.BlockSpec((bm_e, hid), lambda i: (i, 0)),
        out_shape=jax.ShapeDtypeStruct((n, hid), jnp.float32),
    )(x2, W_enc, b_enc2)

    c0 = pl.pallas_call(
        functools.partial(_prop_body, beta=beta1),
        grid=grid,
        in_specs=[pl.BlockSpec((bm, n), lambda i: (i, 0)),
                  full_blk(h0), row_blk(hid)],
        out_specs=row_blk(hid),
        out_shape=jax.ShapeDtypeStruct((n, hid), jnp.float32),
    )(adj, h0, h0)

    logp, z1n, z2n = pl.pallas_call(
        functools.partial(_layer2_body, beta=beta2),
        grid=grid,
        in_specs=[pl.BlockSpec((bm, n), lambda i: (i, 0)),
                  full_blk(c0), row_blk(hid), row_blk(hid), row_blk(hid),
                  full_blk(W_fc), full_blk(b_fc2)],
        out_specs=[row_blk(ncls), row_blk(hid), row_blk(hid)],
        out_shape=[jax.ShapeDtypeStruct((n, ncls), jnp.float32),
                   jax.ShapeDtypeStruct((n, hid), jnp.float32),
                   jax.ShapeDtypeStruct((n, hid), jnp.float32)],
    )(adj, c0, c0, u1, u2, W_fc, b_fc2)

    loss_acc = pl.pallas_call(
        _loss_body,
        grid=grid,
        in_specs=[row_blk(hid), row_blk(hid), full_blk(z1n), full_blk(z2n)],
        out_specs=pl.BlockSpec(memory_space=pltpu.SMEM),
        out_shape=jax.ShapeDtypeStruct((1, 1), jnp.float32),
    )(z1n, z2n, z1n, z2n)

    loss = (loss_acc[0, 0] / n).astype(jnp.float32)
    return (logp, loss, 0, 0)


# speedup vs baseline: 1.9225x; 1.1587x over previous
import functools
import math

import jax
import jax.numpy as jnp
from jax.experimental import pallas as pl
from jax.experimental.pallas import tpu as pltpu

LAMDA = 0.5
TAU = 0.5
P = 0.5
_INV_TAU = 1.0 / TAU
_INV_KEEP = 1.0 / (1.0 - P)


def _threefry_uniform(k0, k1, f):
    x1 = f.astype(jnp.uint32)
    x0 = jnp.zeros_like(x1)
    ks2 = k0 ^ k1 ^ jnp.uint32(0x1BD11BDA)
    ks = (k0, k1, ks2)
    x0 = x0 + k0
    x1 = x1 + k1
    rot_a = (13, 15, 26, 6)
    rot_b = (17, 29, 16, 24)
    for g in range(5):
        for r in (rot_a if g % 2 == 0 else rot_b):
            x0 = x0 + x1
            x1 = (x1 << jnp.uint32(r)) | (x1 >> jnp.uint32(32 - r))
            x1 = x1 ^ x0
        x0 = x0 + ks[(g + 1) % 3]
        x1 = x1 + ks[(g + 2) % 3] + jnp.uint32(g + 1)
    bits = x0 ^ x1
    fbits = (bits >> jnp.uint32(9)) | jnp.uint32(0x3F800000)
    return jax.lax.bitcast_convert_type(fbits, jnp.float32) - 1.0


def _enc_body(x_ref, w_ref, b_ref, out_ref):
    out_ref[...] = jax.lax.dot_general(
        x_ref[...], w_ref[...], (((1,), (1,)), ((), ())),
        preferred_element_type=jnp.float32) + b_ref[...]


def _prop_body(adj_ref, hfull_ref, hrow_ref, out_ref, *, beta):
    hi = jax.lax.dot_general(
        adj_ref[...], hfull_ref[...], (((1,), (0,)), ((), ())),
        preferred_element_type=jnp.float32)
    out_ref[...] = beta * hrow_ref[...] + (1.0 - beta) * hi


def _layer2_body(key_ref, adj_ref, c0full_ref, c0row_ref, wfc_ref,
                 bfc_ref, logp_ref, z1n_ref, z2n_ref, *, beta):
    i = pl.program_id(0)
    bm, hid = c0row_ref.shape

    hi = jax.lax.dot_general(
        adj_ref[...], c0full_ref[...], (((1,), (0,)), ((), ())),
        preferred_element_type=jnp.float32)
    c0row = c0row_ref[...]
    cur1 = beta * c0row + (1.0 - beta) * hi

    base = i * (bm * hid)
    f = (base
         + jax.lax.broadcasted_iota(jnp.int32, (bm, hid), 0) * hid
         + jax.lax.broadcasted_iota(jnp.int32, (bm, hid), 1))
    u1 = _threefry_uniform(key_ref[0, 0], key_ref[0, 1], f)
    u2 = _threefry_uniform(key_ref[1, 0], key_ref[1, 1], f)

    z1 = jnp.where(u1 >= P, cur1 * _INV_KEEP, 0.0)
    z2 = jnp.where(u2 >= P, c0row * _INV_KEEP, 0.0)
    n1 = jnp.sqrt(jnp.sum(z1 * z1, axis=1, keepdims=True))
    n2 = jnp.sqrt(jnp.sum(z2 * z2, axis=1, keepdims=True))
    z1n_ref[...] = (z1 / jnp.maximum(n1, 1e-12)).astype(jnp.bfloat16)
    z2n_ref[...] = (z2 / jnp.maximum(n2, 1e-12)).astype(jnp.bfloat16)

    fc = jax.lax.dot_general(
        cur1, wfc_ref[...], (((1,), (1,)), ((), ())),
        preferred_element_type=jnp.float32) + bfc_ref[...]
    m = jnp.max(fc, axis=1, keepdims=True)
    sh = fc - m
    logp_ref[...] = sh - jnp.log(jnp.sum(jnp.exp(sh), axis=1, keepdims=True))


def _loss_body(z1row_ref, z2row_ref, z1full_ref, z2full_ref, out_ref):
    i = pl.program_id(0)
    a = z1row_ref[...]
    a32 = a.astype(jnp.float32)
    a_sc = a * jnp.bfloat16(_INV_TAU)
    s1 = jax.lax.dot_general(
        a_sc, z1full_ref[...], (((1,), (1,)), ((), ())),
        preferred_element_type=jnp.float32)
    s2 = jax.lax.dot_general(
        a_sc, z2full_ref[...], (((1,), (1,)), ((), ())),
        preferred_element_type=jnp.float32)
    rs = (jnp.sum(jnp.exp(s1), axis=1, keepdims=True)
          + jnp.sum(jnp.exp(s2), axis=1, keepdims=True))
    d_refl = jnp.exp(jnp.sum(a32 * a32, axis=1, keepdims=True) * _INV_TAU)
    b32 = z2row_ref[...].astype(jnp.float32)
    dotb = jnp.sum(a32 * b32, axis=1, keepdims=True)
    d_btw = jnp.exp(dotb * _INV_TAU)
    ct = jnp.log(rs - d_refl - d_btw) - dotb * _INV_TAU
    bs = jnp.sum(ct)

    @pl.when(i == 0)
    def _():
        out_ref[0, 0] = 0.0

    out_ref[0, 0] += bs


def kernel(x, adj, W_enc, b_enc, W_fc, b_fc):
    x2 = jnp.squeeze(x, 0)
    n, nfeat = x2.shape
    hid = W_enc.shape[0]
    ncls = W_fc.shape[0]

    bm = 400 if n % 400 == 0 else n
    bl = 200 if n % 200 == 0 else n
    beta1 = math.log(LAMDA / 1.0 + 1.0)
    beta2 = math.log(LAMDA / 2.0 + 1.0)

    b_enc2 = b_enc.reshape(1, hid)
    b_fc2 = b_fc.reshape(1, ncls)

    dkey = jax.random.key(1)
    k1, k2 = jax.random.split(jax.random.fold_in(dkey, 1))
    keybits = jnp.stack([jax.random.key_data(k1),
                         jax.random.key_data(k2)]).astype(jnp.uint32)

    row_blk = lambda b, w: pl.BlockSpec((b, w), lambda i: (i, 0))
    full_blk = lambda a: pl.BlockSpec(a.shape, lambda i: (0, 0))

    bm_e = 2000 if n % 2000 == 0 else bm
    h0 = pl.pallas_call(
        _enc_body,
        grid=(n // bm_e,),
        in_specs=[pl.BlockSpec((bm_e, nfeat), lambda i: (i, 0)),
                  full_blk(W_enc), full_blk(b_enc2)],
        out_specs=pl.BlockSpec((bm_e, hid), lambda i: (i, 0)),
        out_shape=jax.ShapeDtypeStruct((n, hid), jnp.float32),
    )(x2, W_enc, b_enc2)

    c0 = pl.pallas_call(
        functools.partial(_prop_body, beta=beta1),
        grid=(n // bm,),
        in_specs=[pl.BlockSpec((bm, n), lambda i: (i, 0)),
                  full_blk(h0), row_blk(bm, hid)],
        out_specs=row_blk(bm, hid),
        out_shape=jax.ShapeDtypeStruct((n, hid), jnp.float32),
    )(adj, h0, h0)

    logp, z1n, z2n = pl.pallas_call(
        functools.partial(_layer2_body, beta=beta2),
        grid=(n // bm,),
        in_specs=[pl.BlockSpec(memory_space=pltpu.SMEM),
                  pl.BlockSpec((bm, n), lambda i: (i, 0)),
                  full_blk(c0), row_blk(bm, hid),
                  full_blk(W_fc), full_blk(b_fc2)],
        out_specs=[row_blk(bm, ncls), row_blk(bm, hid), row_blk(bm, hid)],
        out_shape=[jax.ShapeDtypeStruct((n, ncls), jnp.float32),
                   jax.ShapeDtypeStruct((n, hid), jnp.bfloat16),
                   jax.ShapeDtypeStruct((n, hid), jnp.bfloat16)],
    )(keybits, adj, c0, c0, W_fc, b_fc2)

    loss_acc = pl.pallas_call(
        _loss_body,
        grid=(n // bl,),
        in_specs=[row_blk(bl, hid), row_blk(bl, hid),
                  full_blk(z1n), full_blk(z2n)],
        out_specs=pl.BlockSpec(memory_space=pltpu.SMEM),
        out_shape=jax.ShapeDtypeStruct((1, 1), jnp.float32),
    )(z1n, z2n, z1n, z2n)

    loss = (loss_acc[0, 0] / n).astype(jnp.float32)
    return (logp, loss, 0, 0)


# speedup vs baseline: 2.0365x; 1.0593x over previous
import functools
import math

import jax
import jax.numpy as jnp
from jax.experimental import pallas as pl
from jax.experimental.pallas import tpu as pltpu

LAMDA = 0.5
TAU = 0.5
P = 0.5
_INV_TAU = 1.0 / TAU
_INV_KEEP = 1.0 / (1.0 - P)


def _threefry_uniform(k0, k1, f):
    x1 = f.astype(jnp.uint32)
    x0 = jnp.zeros_like(x1)
    ks2 = k0 ^ k1 ^ jnp.uint32(0x1BD11BDA)
    ks = (k0, k1, ks2)
    x0 = x0 + k0
    x1 = x1 + k1
    rot_a = (13, 15, 26, 6)
    rot_b = (17, 29, 16, 24)
    for g in range(5):
        for r in (rot_a if g % 2 == 0 else rot_b):
            x0 = x0 + x1
            x1 = (x1 << jnp.uint32(r)) | (x1 >> jnp.uint32(32 - r))
            x1 = x1 ^ x0
        x0 = x0 + ks[(g + 1) % 3]
        x1 = x1 + ks[(g + 2) % 3] + jnp.uint32(g + 1)
    bits = x0 ^ x1
    fbits = (bits >> jnp.uint32(9)) | jnp.uint32(0x3F800000)
    return jax.lax.bitcast_convert_type(fbits, jnp.float32) - 1.0


def _enc_body(x_ref, w_ref, b_ref, out_ref):
    out_ref[...] = jax.lax.dot_general(
        x_ref[...], w_ref[...], (((1,), (1,)), ((), ())),
        preferred_element_type=jnp.float32) + b_ref[...]


def _prop_body(adja_ref, adjb_ref, hfull_ref, hrow_ref, out_ref, *, beta, bh):
    hf = hfull_ref[...]
    hi_a = jax.lax.dot_general(
        adja_ref[...], hf, (((1,), (0,)), ((), ())),
        preferred_element_type=jnp.float32)
    hi_b = jax.lax.dot_general(
        adjb_ref[...], hf, (((1,), (0,)), ((), ())),
        preferred_element_type=jnp.float32)
    out_ref[:bh, :] = beta * hrow_ref[:bh, :] + (1.0 - beta) * hi_a
    out_ref[bh:, :] = beta * hrow_ref[bh:, :] + (1.0 - beta) * hi_b


def _layer2_body(key_ref, adja_ref, adjb_ref, c0full_ref, c0row_ref, wfc_ref,
                 bfc_ref, logp_ref, z1n_ref, z2n_ref, *, beta):
    i = pl.program_id(0)
    bm, hid = c0row_ref.shape

    cf = c0full_ref[...]
    hi_a = jax.lax.dot_general(
        adja_ref[...], cf, (((1,), (0,)), ((), ())),
        preferred_element_type=jnp.float32)
    hi_b = jax.lax.dot_general(
        adjb_ref[...], cf, (((1,), (0,)), ((), ())),
        preferred_element_type=jnp.float32)
    hi = jnp.concatenate([hi_a, hi_b], axis=0)
    c0row = c0row_ref[...]
    cur1 = beta * c0row + (1.0 - beta) * hi

    base = i * (bm * hid)
    f = (base
         + jax.lax.broadcasted_iota(jnp.int32, (bm, hid), 0) * hid
         + jax.lax.broadcasted_iota(jnp.int32, (bm, hid), 1))
    u1 = _threefry_uniform(key_ref[0, 0], key_ref[0, 1], f)
    u2 = _threefry_uniform(key_ref[1, 0], key_ref[1, 1], f)

    z1 = jnp.where(u1 >= P, cur1 * _INV_KEEP, 0.0)
    z2 = jnp.where(u2 >= P, c0row * _INV_KEEP, 0.0)
    n1 = jnp.sqrt(jnp.sum(z1 * z1, axis=1, keepdims=True))
    n2 = jnp.sqrt(jnp.sum(z2 * z2, axis=1, keepdims=True))
    z1n_ref[...] = (z1 / jnp.maximum(n1, 1e-12)).astype(jnp.bfloat16)
    z2n_ref[...] = (z2 / jnp.maximum(n2, 1e-12)).astype(jnp.bfloat16)

    fc = jax.lax.dot_general(
        cur1, wfc_ref[...], (((1,), (1,)), ((), ())),
        preferred_element_type=jnp.float32) + bfc_ref[...]
    m = jnp.max(fc, axis=1, keepdims=True)
    sh = fc - m
    logp_ref[...] = sh - jnp.log(jnp.sum(jnp.exp(sh), axis=1, keepdims=True))


def _loss_body(z1row_ref, z2row_ref, z1full_ref, z2full_ref, out_ref):
    i = pl.program_id(0)
    a = z1row_ref[...]
    a32 = a.astype(jnp.float32)
    a_sc = a * jnp.bfloat16(_INV_TAU * 1.4426950408889634)
    s1 = jax.lax.dot_general(
        a_sc, z1full_ref[...], (((1,), (1,)), ((), ())),
        preferred_element_type=jnp.float32)
    s2 = jax.lax.dot_general(
        a_sc, z2full_ref[...], (((1,), (1,)), ((), ())),
        preferred_element_type=jnp.float32)
    rs = (jnp.sum(jnp.exp2(s1), axis=1, keepdims=True)
          + jnp.sum(jnp.exp2(s2), axis=1, keepdims=True))
    d_refl = jnp.exp(jnp.sum(a32 * a32, axis=1, keepdims=True) * _INV_TAU)
    b32 = z2row_ref[...].astype(jnp.float32)
    dotb = jnp.sum(a32 * b32, axis=1, keepdims=True)
    d_btw = jnp.exp(dotb * _INV_TAU)
    ct = jnp.log(rs - d_refl - d_btw) - dotb * _INV_TAU
    bs = jnp.sum(ct)

    @pl.when(i == 0)
    def _():
        out_ref[0, 0] = 0.0

    out_ref[0, 0] += bs


def kernel(x, adj, W_enc, b_enc, W_fc, b_fc):
    x2 = jnp.squeeze(x, 0)
    n, nfeat = x2.shape
    hid = W_enc.shape[0]
    ncls = W_fc.shape[0]

    bm = 400 if n % 400 == 0 else n
    bh = bm // 2
    bl = 400 if n % 400 == 0 else n
    beta1 = math.log(LAMDA / 1.0 + 1.0)
    beta2 = math.log(LAMDA / 2.0 + 1.0)

    b_enc2 = b_enc.reshape(1, hid)
    b_fc2 = b_fc.reshape(1, ncls)

    dkey = jax.random.key(1)
    k1, k2 = jax.random.split(jax.random.fold_in(dkey, 1))
    keybits = jnp.stack([jax.random.key_data(k1),
                         jax.random.key_data(k2)]).astype(jnp.uint32)

    row_blk = lambda b, w: pl.BlockSpec((b, w), lambda i: (i, 0))
    full_blk = lambda a: pl.BlockSpec(a.shape, lambda i: (0, 0))

    bm_e = 2000 if n % 2000 == 0 else bm
    h0 = pl.pallas_call(
        _enc_body,
        grid=(n // bm_e,),
        in_specs=[pl.BlockSpec((bm_e, nfeat), lambda i: (i, 0)),
                  full_blk(W_enc), full_blk(b_enc2)],
        out_specs=pl.BlockSpec((bm_e, hid), lambda i: (i, 0)),
        out_shape=jax.ShapeDtypeStruct((n, hid), jnp.float32),
    )(x2, W_enc, b_enc2)

    c0 = pl.pallas_call(
        functools.partial(_prop_body, beta=beta1, bh=bh),
        grid=(n // bm,),
        in_specs=[pl.BlockSpec((bh, n), lambda i: (2 * i, 0)),
                  pl.BlockSpec((bh, n), lambda i: (2 * i + 1, 0)),
                  full_blk(h0), row_blk(bm, hid)],
        out_specs=row_blk(bm, hid),
        out_shape=jax.ShapeDtypeStruct((n, hid), jnp.float32),
    )(adj, adj, h0, h0)

    logp, z1n, z2n = pl.pallas_call(
        functools.partial(_layer2_body, beta=beta2),
        grid=(n // bm,),
        in_specs=[pl.BlockSpec(memory_space=pltpu.SMEM),
                  pl.BlockSpec((bh, n), lambda i: (2 * i, 0)),
                  pl.BlockSpec((bh, n), lambda i: (2 * i + 1, 0)),
                  full_blk(c0), row_blk(bm, hid),
                  full_blk(W_fc), full_blk(b_fc2)],
        out_specs=[row_blk(bm, ncls), row_blk(bm, hid), row_blk(bm, hid)],
        out_shape=[jax.ShapeDtypeStruct((n, ncls), jnp.float32),
                   jax.ShapeDtypeStruct((n, hid), jnp.bfloat16),
                   jax.ShapeDtypeStruct((n, hid), jnp.bfloat16)],
    )(keybits, adj, adj, c0, c0, W_fc, b_fc2)

    loss_acc = pl.pallas_call(
        _loss_body,
        grid=(n // bl,),
        in_specs=[row_blk(bl, hid), row_blk(bl, hid),
                  full_blk(z1n), full_blk(z2n)],
        out_specs=pl.BlockSpec(memory_space=pltpu.SMEM),
        out_shape=jax.ShapeDtypeStruct((1, 1), jnp.float32),
    )(z1n, z2n, z1n, z2n)

    loss = (loss_acc[0, 0] / n).astype(jnp.float32)
    return (logp, loss, 0, 0)
